# baseline (speedup 1.0000x reference)
import functools

import jax
import jax.numpy as jnp
from jax import lax
from jax.experimental import pallas as pl
from jax.experimental.pallas import tpu as pltpu
from jax.experimental.pallas import tpu_sc as plsc

N, Q, K = 10000, 128, 10
BM1 = 2000

BM2 = 400

NW = 32
NCHK = N // 16
CPT = 20


def _stage1_body(z_ref, wtT_ref, bt_ref, lnw_ref, lnb_ref, woT_ref, bo_ref,
                 p_ref, pbf_ref, x_ref, colsum_ref, ztp_ref):
    step = pl.program_id(0)
    nsteps = pl.num_programs(0)

    z = z_ref[...]
    m = jnp.dot(z, wtT_ref[...], preferred_element_type=jnp.float32)
    m = m + bt_ref[...]
    mu = jnp.mean(m, axis=1, keepdims=True)
    var = jnp.mean((m - mu) * (m - mu), axis=1, keepdims=True)
    mn = (m - mu) / jnp.sqrt(var + 1e-5) * lnw_ref[...] + lnb_ref[...]
    h = jnp.where(mn >= 0, mn, 0.2 * mn)
    ol = jnp.dot(h, woT_ref[...], preferred_element_type=jnp.float32)
    ol = ol + bo_ref[...]
    olmax = jnp.max(ol, axis=1, keepdims=True)
    e = jnp.exp(ol - olmax)
    p = e / jnp.sum(e, axis=1, keepdims=True)
    p_ref[...] = p[:, :K]
    pbf_ref[...] = p.astype(jnp.bfloat16)

    @pl.when(step == 0)
    def _init():
        colsum_ref[...] = jnp.zeros_like(colsum_ref)
        ztp_ref[...] = jnp.zeros_like(ztp_ref)

    colsum_ref[...] += jnp.sum(p, axis=0, keepdims=True)
    ztp_ref[...] += lax.dot_general(z, p, (((0,), (0,)), ((), ())),
                                    preferred_element_type=jnp.float32)

    @pl.when(step == nsteps - 1)
    def _fin():
        cs = colsum_ref[...]
        lane1 = lax.broadcasted_iota(jnp.int32, cs.shape, 1)
        d = jnp.where(lane1 < K, 1.0 / cs + 1e-8, 0.0)
        x_ref[...] = (ztp_ref[...] * d).T[:K, :]


def _sc_argmax_body(pt_hbm, s_hbm, pcl, sbuf, semp, semo):
    sid = lax.axis_index("s")
    cid = lax.axis_index("c")
    wid = sid * 2 + cid
    nch = jnp.where(wid < 17, CPT, CPT - 1)
    chunk0 = wid * (CPT - 1) + jnp.minimum(wid, 17)
    row0 = chunk0 * 16

    for c in range(K):
        pltpu.async_copy(
            pt_hbm.at[pl.ds(c * N + row0, CPT * 16)],
            pcl.at[pl.ds(c * CPT * 16, CPT * 16)], semp).wait()

    def body(jj, _):
        off = jj * 16
        best_v = pcl[pl.ds(off, 16)]
        best_i = jnp.zeros((16,), jnp.int32)
        for c in range(1, K):
            v = pcl[pl.ds(c * CPT * 16 + off, 16)]
            upd = v > best_v
            best_i = jnp.where(upd, c, best_i)
            best_v = jnp.where(upd, v, best_v)
        sbuf[pl.ds(off, 16)] = best_i
        return 0

    lax.fori_loop(0, nch, body, 0)

    @pl.when(wid < 17)
    def _():
        cp = pltpu.make_async_copy(
            sbuf.at[pl.ds(0, CPT * 16)],
            s_hbm.at[pl.ds(row0, CPT * 16)], semo)
        cp.start()
        cp.wait()

    @pl.when(wid >= 17)
    def _():
        cp = pltpu.make_async_copy(
            sbuf.at[pl.ds(0, (CPT - 1) * 16)],
            s_hbm.at[pl.ds(row0, (CPT - 1) * 16)], semo)
        cp.start()
        cp.wait()


_sc_argmax = functools.partial(
    pl.kernel,
    mesh=plsc.VectorSubcoreMesh(core_axis_name="c", subcore_axis_name="s"),
    out_type=jax.ShapeDtypeStruct((N,), jnp.int32),
    scratch_types=[
        pltpu.VMEM((K * CPT * 16,), jnp.float32),
        pltpu.VMEM((CPT * 16,), jnp.int32),
        pltpu.SemaphoreType.DMA,
        pltpu.SemaphoreType.DMA,
    ],
)(_sc_argmax_body)


def _stage2_body(a_ref, pbf_ref, pblk_ref, at_ref, acc_ref):
    step = pl.program_id(0)
    nsteps = pl.num_programs(0)

    a_bf = a_ref[...].astype(jnp.bfloat16)
    ap = jnp.dot(a_bf, pbf_ref[...], preferred_element_type=jnp.float32)

    @pl.when(step == 0)
    def _init():
        acc_ref[...] = jnp.zeros_like(acc_ref)

    pblk = pblk_ref[...].astype(jnp.float32)
    acc_ref[...] += lax.dot_general(pblk, ap, (((0,), (0,)), ((), ())),
                                    preferred_element_type=jnp.float32)

    @pl.when(step == nsteps - 1)
    def _fin():
        at_ref[...] = acc_ref[...]


def kernel(Z, A, W_t, b_t, ln_w, ln_b, W_o, b_o):
    wtT = W_t.T
    bt = b_t.reshape(1, Q)
    lnw = ln_w.reshape(1, Q)
    lnb = ln_b.reshape(1, Q)
    woT = jnp.zeros((Q, 128), jnp.float32).at[:, :K].set(W_o.T)
    bo = jnp.full((1, 128), -1e30, jnp.float32).at[0, :K].set(b_o)

    grid1 = N // BM1
    P, p_bf, X_tilde = pl.pallas_call(
        _stage1_body,
        grid=(grid1,),
        in_specs=[
            pl.BlockSpec((BM1, Q), lambda i: (i, 0)),
            pl.BlockSpec((Q, Q), lambda i: (0, 0)),
            pl.BlockSpec((1, Q), lambda i: (0, 0)),
            pl.BlockSpec((1, Q), lambda i: (0, 0)),
            pl.BlockSpec((1, Q), lambda i: (0, 0)),
            pl.BlockSpec((Q, 128), lambda i: (0, 0)),
            pl.BlockSpec((1, 128), lambda i: (0, 0)),
        ],
        out_specs=[
            pl.BlockSpec((BM1, K), lambda i: (i, 0)),
            pl.BlockSpec((BM1, 128), lambda i: (i, 0)),
            pl.BlockSpec((K, Q), lambda i: (0, 0)),
        ],
        out_shape=[
            jax.ShapeDtypeStruct((N, K), jnp.float32),
            jax.ShapeDtypeStruct((N, 128), jnp.bfloat16),
            jax.ShapeDtypeStruct((K, Q), jnp.float32),
        ],
        scratch_shapes=[
            pltpu.VMEM((1, 128), jnp.float32),
            pltpu.VMEM((128, 128), jnp.float32),
        ],
    )(Z, wtT, bt, lnw, lnb, woT, bo)

    pt32 = jnp.pad(P.T.reshape(-1), (0, 128))

    S = _sc_argmax(pt32)

    at_full = pl.pallas_call(
        _stage2_body,
        grid=(N // BM2,),
        in_specs=[
            pl.BlockSpec((BM2, N), lambda i: (i, 0)),
            pl.BlockSpec((N, 128), lambda i: (0, 0)),
            pl.BlockSpec((BM2, 128), lambda i: (i, 0)),
        ],
        out_specs=pl.BlockSpec((128, 128), lambda i: (0, 0)),
        out_shape=jax.ShapeDtypeStruct((128, 128), jnp.float32),
        scratch_shapes=[pltpu.VMEM((128, 128), jnp.float32)],
    )(A, p_bf, p_bf)

    A_tilde = at_full[:K, :K]
    return X_tilde, A_tilde, P, S


# speedup vs baseline: 1.0019x; 1.0019x over previous
import functools

import jax
import jax.numpy as jnp
from jax import lax
from jax.experimental import pallas as pl
from jax.experimental.pallas import tpu as pltpu
from jax.experimental.pallas import tpu_sc as plsc

N, Q, K = 10000, 128, 10
BM1 = 2000

BM2 = 400

NW = 32
NCHK = N // 16
CPT = 20


def _stage1_body(z_ref, wtT_ref, bt_ref, lnw_ref, lnb_ref, woT_ref, bo_ref,
                 p_ref, pbf_ref, x_ref, colsum_ref, ztp_ref):
    step = pl.program_id(0)
    nsteps = pl.num_programs(0)

    z = z_ref[...]
    m = jnp.dot(z, wtT_ref[...], preferred_element_type=jnp.float32)
    m = m + bt_ref[...]
    mu = jnp.mean(m, axis=1, keepdims=True)
    var = jnp.mean((m - mu) * (m - mu), axis=1, keepdims=True)
    mn = (m - mu) / jnp.sqrt(var + 1e-5) * lnw_ref[...] + lnb_ref[...]
    h = jnp.where(mn >= 0, mn, 0.2 * mn)
    ol = jnp.dot(h, woT_ref[...], preferred_element_type=jnp.float32)
    ol = ol + bo_ref[...]
    olmax = jnp.max(ol, axis=1, keepdims=True)
    e = jnp.exp(ol - olmax)
    p = e / jnp.sum(e, axis=1, keepdims=True)
    p_ref[...] = p[:, :K]
    pbf_ref[...] = p.astype(jnp.bfloat16)

    @pl.when(step == 0)
    def _init():
        colsum_ref[...] = jnp.zeros_like(colsum_ref)
        ztp_ref[...] = jnp.zeros_like(ztp_ref)

    colsum_ref[...] += jnp.sum(p, axis=0, keepdims=True)
    ztp_ref[...] += lax.dot_general(z, p, (((0,), (0,)), ((), ())),
                                    preferred_element_type=jnp.float32)

    @pl.when(step == nsteps - 1)
    def _fin():
        cs = colsum_ref[...]
        lane1 = lax.broadcasted_iota(jnp.int32, cs.shape, 1)
        d = jnp.where(lane1 < K, 1.0 / cs + 1e-8, 0.0)
        x_ref[...] = (ztp_ref[...] * d).T[:K, :]


def _sc_argmax_body(pt_hbm, s_hbm, pcl, sbuf, semp, semo):
    sid = lax.axis_index("s")
    cid = lax.axis_index("c")
    wid = sid * 2 + cid
    nch = jnp.where(wid < 17, CPT, CPT - 1)
    chunk0 = wid * (CPT - 1) + jnp.minimum(wid, 17)
    row0 = chunk0 * 16

    for c in range(K):
        pltpu.async_copy(
            pt_hbm.at[pl.ds(c * N + row0, CPT * 16)],
            pcl.at[pl.ds(c * CPT * 16, CPT * 16)], semp).wait()

    def body(jj, _):
        off = jj * 16
        best_v = pcl[pl.ds(off, 16)]
        best_i = jnp.zeros((16,), jnp.int32)
        for c in range(1, K):
            v = pcl[pl.ds(c * CPT * 16 + off, 16)]
            upd = v > best_v
            best_i = jnp.where(upd, c, best_i)
            best_v = jnp.where(upd, v, best_v)
        sbuf[pl.ds(off, 16)] = best_i
        return 0

    lax.fori_loop(0, nch, body, 0)

    @pl.when(wid < 17)
    def _():
        cp = pltpu.make_async_copy(
            sbuf.at[pl.ds(0, CPT * 16)],
            s_hbm.at[pl.ds(row0, CPT * 16)], semo)
        cp.start()
        cp.wait()

    @pl.when(wid >= 17)
    def _():
        cp = pltpu.make_async_copy(
            sbuf.at[pl.ds(0, (CPT - 1) * 16)],
            s_hbm.at[pl.ds(row0, (CPT - 1) * 16)], semo)
        cp.start()
        cp.wait()


_sc_argmax = functools.partial(
    pl.kernel,
    mesh=plsc.VectorSubcoreMesh(core_axis_name="c", subcore_axis_name="s"),
    out_type=jax.ShapeDtypeStruct((N,), jnp.int32),
    scratch_types=[
        pltpu.VMEM((K * CPT * 16,), jnp.float32),
        pltpu.VMEM((CPT * 16,), jnp.int32),
        pltpu.SemaphoreType.DMA,
        pltpu.SemaphoreType.DMA,
    ],
)(_sc_argmax_body)


def _stage2_body(a_ref, pbf_ref, pblk_ref, at_ref, acc_ref):
    step = pl.program_id(0)
    nsteps = pl.num_programs(0)

    a_bf = a_ref[...].astype(jnp.bfloat16)
    ap = jnp.dot(a_bf, pbf_ref[...], preferred_element_type=jnp.float32)

    @pl.when(step == 0)
    def _init():
        acc_ref[...] = jnp.zeros_like(acc_ref)

    pblk = pblk_ref[...].astype(jnp.float32)
    acc_ref[...] += lax.dot_general(pblk, ap, (((0,), (0,)), ((), ())),
                                    preferred_element_type=jnp.float32)

    @pl.when(step == nsteps - 1)
    def _fin():
        at_ref[...] = acc_ref[...]


def kernel(Z, A, W_t, b_t, ln_w, ln_b, W_o, b_o):
    wtT = W_t.T
    bt = b_t.reshape(1, Q)
    lnw = ln_w.reshape(1, Q)
    lnb = ln_b.reshape(1, Q)
    woT = jnp.zeros((Q, 128), jnp.float32).at[:, :K].set(W_o.T)
    bo = jnp.full((1, 128), -1e30, jnp.float32).at[0, :K].set(b_o)

    grid1 = N // BM1
    P, p_bf, X_tilde = pl.pallas_call(
        _stage1_body,
        grid=(grid1,),
        in_specs=[
            pl.BlockSpec((BM1, Q), lambda i: (i, 0)),
            pl.BlockSpec((Q, Q), lambda i: (0, 0)),
            pl.BlockSpec((1, Q), lambda i: (0, 0)),
            pl.BlockSpec((1, Q), lambda i: (0, 0)),
            pl.BlockSpec((1, Q), lambda i: (0, 0)),
            pl.BlockSpec((Q, 128), lambda i: (0, 0)),
            pl.BlockSpec((1, 128), lambda i: (0, 0)),
        ],
        out_specs=[
            pl.BlockSpec((BM1, K), lambda i: (i, 0)),
            pl.BlockSpec((BM1, 128), lambda i: (i, 0)),
            pl.BlockSpec((K, Q), lambda i: (0, 0)),
        ],
        out_shape=[
            jax.ShapeDtypeStruct((N, K), jnp.float32),
            jax.ShapeDtypeStruct((N, 128), jnp.bfloat16),
            jax.ShapeDtypeStruct((K, Q), jnp.float32),
        ],
        scratch_shapes=[
            pltpu.VMEM((1, 128), jnp.float32),
            pltpu.VMEM((128, 128), jnp.float32),
        ],
    )(Z, wtT, bt, lnw, lnb, woT, bo)

    pt32 = jnp.pad(P.T.reshape(-1), (0, 128))

    at_full = pl.pallas_call(
        _stage2_body,
        grid=(N // BM2,),
        in_specs=[
            pl.BlockSpec((BM2, N), lambda i: (i, 0)),
            pl.BlockSpec((N, 128), lambda i: (0, 0)),
            pl.BlockSpec((BM2, 128), lambda i: (i, 0)),
        ],
        out_specs=pl.BlockSpec((128, 128), lambda i: (0, 0)),
        out_shape=jax.ShapeDtypeStruct((128, 128), jnp.float32),
        scratch_shapes=[pltpu.VMEM((128, 128), jnp.float32)],
    )(A, p_bf, p_bf)

    S = _sc_argmax(pt32)

    A_tilde = at_full[:K, :K]
    return X_tilde, A_tilde, P, S


# speedup vs baseline: 1.0139x; 1.0119x over previous
import functools

import jax
import jax.numpy as jnp
from jax import lax
from jax.experimental import pallas as pl
from jax.experimental.pallas import tpu as pltpu
from jax.experimental.pallas import tpu_sc as plsc

N, Q, K = 10000, 128, 10
BM1 = 2000

BM2 = 400

NW = 32
NCHK = N // 16
CPT = 20


def _stage1_body(z_ref, wtT_ref, bt_ref, lnw_ref, lnb_ref, woT_ref, bo_ref,
                 p_ref, pbf_ref, x_ref, colsum_ref, ztp_ref):
    step = pl.program_id(0)
    nsteps = pl.num_programs(0)

    z = z_ref[...]
    m = jnp.dot(z, wtT_ref[...], preferred_element_type=jnp.float32)
    m = m + bt_ref[...]
    mu = jnp.mean(m, axis=1, keepdims=True)
    var = jnp.mean((m - mu) * (m - mu), axis=1, keepdims=True)
    mn = (m - mu) / jnp.sqrt(var + 1e-5) * lnw_ref[...] + lnb_ref[...]
    h = jnp.where(mn >= 0, mn, 0.2 * mn)
    ol = jnp.dot(h, woT_ref[...], preferred_element_type=jnp.float32)
    ol = ol + bo_ref[...]
    olmax = jnp.max(ol, axis=1, keepdims=True)
    e = jnp.exp(ol - olmax)
    p = e / jnp.sum(e, axis=1, keepdims=True)
    p_ref[...] = p[:, :K]
    pbf_ref[...] = p.astype(jnp.bfloat16)

    @pl.when(step == 0)
    def _init():
        colsum_ref[...] = jnp.zeros_like(colsum_ref)
        ztp_ref[...] = jnp.zeros_like(ztp_ref)

    colsum_ref[...] += jnp.sum(p, axis=0, keepdims=True)
    ztp_ref[...] += lax.dot_general(z, p, (((0,), (0,)), ((), ())),
                                    preferred_element_type=jnp.float32)

    @pl.when(step == nsteps - 1)
    def _fin():
        cs = colsum_ref[...]
        lane1 = lax.broadcasted_iota(jnp.int32, cs.shape, 1)
        d = jnp.where(lane1 < K, 1.0 / cs + 1e-8, 0.0)
        x_ref[...] = (ztp_ref[...] * d).T[:K, :]


def _sc_argmax_body(pt_hbm, s_hbm, pcl, sbuf, semp, semo):
    sid = lax.axis_index("s")
    cid = lax.axis_index("c")
    wid = sid * 2 + cid
    nch = jnp.where(wid < 17, CPT, CPT - 1)
    chunk0 = wid * (CPT - 1) + jnp.minimum(wid, 17)
    row0 = chunk0 * 16

    cps = [pltpu.make_async_copy(
        pt_hbm.at[pl.ds(c * N + row0, CPT * 16)],
        pcl.at[pl.ds(c * CPT * 16, CPT * 16)], semp) for c in range(K)]
    for cp in cps:
        cp.start()
    for cp in cps:
        cp.wait()

    def body(jj, _):
        off = jj * 16
        best_v = pcl[pl.ds(off, 16)]
        best_i = jnp.zeros((16,), jnp.int32)
        for c in range(1, K):
            v = pcl[pl.ds(c * CPT * 16 + off, 16)]
            upd = v > best_v
            best_i = jnp.where(upd, c, best_i)
            best_v = jnp.where(upd, v, best_v)
        sbuf[pl.ds(off, 16)] = best_i
        return 0

    lax.fori_loop(0, nch, body, 0)

    @pl.when(wid < 17)
    def _():
        cp = pltpu.make_async_copy(
            sbuf.at[pl.ds(0, CPT * 16)],
            s_hbm.at[pl.ds(row0, CPT * 16)], semo)
        cp.start()
        cp.wait()

    @pl.when(wid >= 17)
    def _():
        cp = pltpu.make_async_copy(
            sbuf.at[pl.ds(0, (CPT - 1) * 16)],
            s_hbm.at[pl.ds(row0, (CPT - 1) * 16)], semo)
        cp.start()
        cp.wait()


_sc_argmax = functools.partial(
    pl.kernel,
    mesh=plsc.VectorSubcoreMesh(core_axis_name="c", subcore_axis_name="s"),
    out_type=jax.ShapeDtypeStruct((N,), jnp.int32),
    scratch_types=[
        pltpu.VMEM((K * CPT * 16,), jnp.float32),
        pltpu.VMEM((CPT * 16,), jnp.int32),
        pltpu.SemaphoreType.DMA,
        pltpu.SemaphoreType.DMA,
    ],
)(_sc_argmax_body)


def _stage2_body(a_ref, pbf_ref, pblk_ref, at_ref, acc_ref):
    step = pl.program_id(0)
    nsteps = pl.num_programs(0)

    a_bf = a_ref[...].astype(jnp.bfloat16)
    ap = jnp.dot(a_bf, pbf_ref[...], preferred_element_type=jnp.float32)

    @pl.when(step == 0)
    def _init():
        acc_ref[...] = jnp.zeros_like(acc_ref)

    pblk = pblk_ref[...].astype(jnp.float32)
    acc_ref[...] += lax.dot_general(pblk, ap, (((0,), (0,)), ((), ())),
                                    preferred_element_type=jnp.float32)

    @pl.when(step == nsteps - 1)
    def _fin():
        at_ref[...] = acc_ref[...]


def kernel(Z, A, W_t, b_t, ln_w, ln_b, W_o, b_o):
    wtT = W_t.T
    bt = b_t.reshape(1, Q)
    lnw = ln_w.reshape(1, Q)
    lnb = ln_b.reshape(1, Q)
    woT = jnp.zeros((Q, 128), jnp.float32).at[:, :K].set(W_o.T)
    bo = jnp.full((1, 128), -1e30, jnp.float32).at[0, :K].set(b_o)

    grid1 = N // BM1
    P, p_bf, X_tilde = pl.pallas_call(
        _stage1_body,
        grid=(grid1,),
        in_specs=[
            pl.BlockSpec((BM1, Q), lambda i: (i, 0)),
            pl.BlockSpec((Q, Q), lambda i: (0, 0)),
            pl.BlockSpec((1, Q), lambda i: (0, 0)),
            pl.BlockSpec((1, Q), lambda i: (0, 0)),
            pl.BlockSpec((1, Q), lambda i: (0, 0)),
            pl.BlockSpec((Q, 128), lambda i: (0, 0)),
            pl.BlockSpec((1, 128), lambda i: (0, 0)),
        ],
        out_specs=[
            pl.BlockSpec((BM1, K), lambda i: (i, 0)),
            pl.BlockSpec((BM1, 128), lambda i: (i, 0)),
            pl.BlockSpec((K, Q), lambda i: (0, 0)),
        ],
        out_shape=[
            jax.ShapeDtypeStruct((N, K), jnp.float32),
            jax.ShapeDtypeStruct((N, 128), jnp.bfloat16),
            jax.ShapeDtypeStruct((K, Q), jnp.float32),
        ],
        scratch_shapes=[
            pltpu.VMEM((1, 128), jnp.float32),
            pltpu.VMEM((128, 128), jnp.float32),
        ],
    )(Z, wtT, bt, lnw, lnb, woT, bo)

    pt32 = jnp.pad(P.T.reshape(-1), (0, 128))

    at_full = pl.pallas_call(
        _stage2_body,
        grid=(N // BM2,),
        in_specs=[
            pl.BlockSpec((BM2, N), lambda i: (i, 0)),
            pl.BlockSpec((N, 128), lambda i: (0, 0)),
            pl.BlockSpec((BM2, 128), lambda i: (i, 0)),
        ],
        out_specs=pl.BlockSpec((128, 128), lambda i: (0, 0)),
        out_shape=jax.ShapeDtypeStruct((128, 128), jnp.float32),
        scratch_shapes=[pltpu.VMEM((128, 128), jnp.float32)],
    )(A, p_bf, p_bf)

    S = _sc_argmax(pt32)

    A_tilde = at_full[:K, :K]
    return X_tilde, A_tilde, P, S
